# Initial kernel scaffold; baseline (speedup 1.0000x reference)
import jax
import jax.numpy as jnp
from jax.experimental import pallas as pl


def kernel(x_data, table):
    raise NotImplementedError("write your pallas kernel here")


# speedup vs baseline: 1.4090x; 1.4090x over previous
import functools

import jax
import jax.numpy as jnp
from jax import lax
from jax.experimental import pallas as pl
from jax.experimental.pallas import tpu as pltpu
from jax.experimental.pallas import tpu_sc as plsc

NC = 2
NS = 16
NW = NC * NS

CHUNK = 2048


def _gather_body(idx_hbm, table_hbm, out_hbm, idx_v, rows_v, sem, *, b_per_w, n_chunks):
    wid = lax.axis_index("s") * NC + lax.axis_index("c")
    base = wid * b_per_w

    def body(i, _):
        off = base + i * CHUNK
        pltpu.sync_copy(idx_hbm.at[pl.ds(off, CHUNK)], idx_v)
        pltpu.async_copy(table_hbm.at[idx_v], rows_v, sem).wait()
        pltpu.sync_copy(rows_v, out_hbm.at[pl.ds(off, CHUNK)])
        return ()

    lax.fori_loop(0, n_chunks, body, ())


def kernel(x_data, table):
    (B,) = x_data.shape
    V, D = table.shape
    assert B % (NW * CHUNK) == 0
    b_per_w = B // NW
    n_chunks = b_per_w // CHUNK

    mesh = plsc.VectorSubcoreMesh(core_axis_name="c", subcore_axis_name="s")
    gather = functools.partial(
        _gather_body, b_per_w=b_per_w, n_chunks=n_chunks
    )
    run = pl.kernel(
        gather,
        out_type=jax.ShapeDtypeStruct((B, D), jnp.float32),
        mesh=mesh,
        scratch_types=[
            pltpu.VMEM((CHUNK,), jnp.int32),
            pltpu.VMEM((CHUNK, D), jnp.float32),
            pltpu.SemaphoreType.DMA,
        ],
        compiler_params=pltpu.CompilerParams(use_tc_tiling_on_sc=False),
    )
    return run(x_data.astype(jnp.int32), table)


# speedup vs baseline: 1.4273x; 1.0130x over previous
import functools

import jax
import jax.numpy as jnp
from jax import lax
from jax.experimental import pallas as pl
from jax.experimental.pallas import tpu as pltpu
from jax.experimental.pallas import tpu_sc as plsc

NC = 2
NS = 16
NW = NC * NS

CHUNK = 1600


def _gather_body(idx_hbm, table_hbm, out_hbm,
                 idx0, idx1, rows0, rows1,
                 s_i0, s_i1, s_g, s_o0, s_o1,
                 *, b_per_w, n_chunks):
    wid = lax.axis_index("s") * NC + lax.axis_index("c")
    base = wid * b_per_w
    n_pairs = n_chunks // 2
    bufs = ((idx0, rows0, s_i0, s_o0), (idx1, rows1, s_i1, s_o1))

    def process(off, b, first, last):
        idx_v, rows_v, s_i, s_o = bufs[b]
        if not first:
            pltpu.make_async_copy(rows_v, out_hbm.at[pl.ds(off, CHUNK)], s_o).wait()
        pltpu.make_async_copy(idx_hbm.at[pl.ds(off, CHUNK)], idx_v, s_i).wait()
        pltpu.async_copy(table_hbm.at[idx_v], rows_v, s_g).wait()
        if not last:
            pltpu.async_copy(idx_hbm.at[pl.ds(off + 2 * CHUNK, CHUNK)], idx_v, s_i)
        pltpu.async_copy(rows_v, out_hbm.at[pl.ds(off, CHUNK)], s_o)

    pltpu.async_copy(idx_hbm.at[pl.ds(base, CHUNK)], idx0, s_i0)
    pltpu.async_copy(idx_hbm.at[pl.ds(base + CHUNK, CHUNK)], idx1, s_i1)

    process(base, 0, True, False)
    process(base + CHUNK, 1, True, False)

    def pair(p, _):
        off = base + (2 * p) * CHUNK
        process(off, 0, False, False)
        process(off + CHUNK, 1, False, False)
        return ()

    lax.fori_loop(1, n_pairs - 1, pair, ())

    tail = base + (n_chunks - 2) * CHUNK
    process(tail, 0, False, True)
    process(tail + CHUNK, 1, False, True)

    pltpu.make_async_copy(rows0, out_hbm.at[pl.ds(tail, CHUNK)], s_o0).wait()
    pltpu.make_async_copy(rows1, out_hbm.at[pl.ds(tail + CHUNK, CHUNK)], s_o1).wait()


def kernel(x_data, table):
    (B,) = x_data.shape
    V, D = table.shape
    assert B % (NW * CHUNK) == 0
    b_per_w = B // NW
    n_chunks = b_per_w // CHUNK
    assert n_chunks % 2 == 0 and n_chunks >= 6

    mesh = plsc.VectorSubcoreMesh(core_axis_name="c", subcore_axis_name="s")
    gather = functools.partial(_gather_body, b_per_w=b_per_w, n_chunks=n_chunks)
    run = pl.kernel(
        gather,
        out_type=jax.ShapeDtypeStruct((B, D), jnp.float32),
        mesh=mesh,
        scratch_types=[
            pltpu.VMEM((CHUNK,), jnp.int32),
            pltpu.VMEM((CHUNK,), jnp.int32),
            pltpu.VMEM((CHUNK, D), jnp.float32),
            pltpu.VMEM((CHUNK, D), jnp.float32),
            pltpu.SemaphoreType.DMA,
            pltpu.SemaphoreType.DMA,
            pltpu.SemaphoreType.DMA,
            pltpu.SemaphoreType.DMA,
            pltpu.SemaphoreType.DMA,
        ],
        compiler_params=pltpu.CompilerParams(use_tc_tiling_on_sc=False),
    )
    return run(x_data.astype(jnp.int32), table)
